# baseline (speedup 1.0000x reference)
import dataclasses
import functools

import jax
import jax.numpy as jnp
from jax import lax
from jax.experimental import pallas as pl
from jax.experimental.pallas import tpu as pltpu
from jax.experimental.pallas import tpu_sc as plsc

_N = 10000
_K = 8
_BITS = 128
_B = 64
_T = 8
_OUT = 10

_NTC = 10240
_NBLK = _NTC // 1024
_NSC = 10240
_NW = 32
_NPW = _NSC // 16
_OUTP = 128


def _pack_body(lutT_ref, x_ref, ini_ref, lutp_ref, xp_ref, sp0_ref):
    rows = lutT_ref[...]
    outs = []
    for j in range(8):
        acc = rows[32 * j]
        for s in range(1, 32):
            acc = acc | (rows[32 * j + s] << s)
        outs.append(acc)
    lutp_ref[...] = jnp.stack(outs)

    neg = -ini_ref[...]
    val = jnp.broadcast_to(neg, (2, 1024))

    @pl.when(pl.program_id(0) == 0)
    def _():
        xr = x_ref[...]
        lo = xr[0]
        hi = xr[32]
        for bb in range(1, 32):
            lo = lo | (xr[bb] << bb)
            hi = hi | (xr[32 + bb] << bb)
        xp_ref[...] = jnp.stack([lo, hi])
        x0 = jnp.stack([lo[0], hi[0]])
        xq = jnp.concatenate([x0, jnp.zeros((2, 1024 - _BITS), jnp.int32)],
                             axis=1)
        sp0_ref[...] = val ^ xq

    @pl.when(pl.program_id(0) != 0)
    def _():
        sp0_ref[...] = val


def _pack_call(lutT3, x3, ini):
    return pl.pallas_call(
        _pack_body,
        grid=(_NBLK,),
        in_specs=[
            pl.BlockSpec((256, 8, 128), lambda i: (0, i, 0)),
            pl.BlockSpec((64, 8, 128), lambda i: (0, 0, 0)),
            pl.BlockSpec((1, 1024), lambda i: (0, i)),
        ],
        out_specs=[
            pl.BlockSpec((8, 8, 128), lambda i: (0, i, 0)),
            pl.BlockSpec((2, 8, 128), lambda i: (0, 0, 0)),
            pl.BlockSpec((2, 1024), lambda i: (0, i)),
        ],
        out_shape=[
            jax.ShapeDtypeStruct((8, _NBLK * 8, 128), jnp.int32),
            jax.ShapeDtypeStruct((2, 8, 128), jnp.int32),
            jax.ShapeDtypeStruct((2, _NTC), jnp.int32),
        ],
    )(lutT3, x3, ini)


def _sc_gather_body(sp_hbm, adj_hbm, out_hbm, sp_v, adj_v, out_v, sp_sh, sem,
                    sem_a):
    c = lax.axis_index("c")
    s = lax.axis_index("s")
    wid = s * 2 + c
    w = wid & 1
    nslice = wid >> 1
    base = nslice * _NPW
    cp_adj = pltpu.async_copy(adj_hbm.at[nslice], adj_v, sem_a)

    @pl.when(s == 0)
    def _():
        pltpu.async_copy(sp_hbm, sp_sh, sem).wait()

    plsc.subcore_barrier()
    pltpu.async_copy(sp_sh.at[w], sp_v, sem).wait()
    cp_adj.wait()

    _H = 384

    @plsc.parallel_loop(0, _H, step=16, unroll=4)
    def _(g):
        for k in range(_K):
            idx = adj_v[k, pl.ds(g, 16)]
            out_v[k, pl.ds(g, 16)] = plsc.load_gather(sp_v, [idx])

    cp_h1 = pltpu.async_copy(out_v.at[:, pl.ds(0, _H)],
                             out_hbm.at[w, :, pl.ds(base, _H)], sem_a)

    @plsc.parallel_loop(_H, _NPW, step=16, unroll=4)
    def _(g):
        for k in range(_K):
            idx = adj_v[k, pl.ds(g, 16)]
            out_v[k, pl.ds(g, 16)] = plsc.load_gather(sp_v, [idx])

    cp_h2 = pltpu.async_copy(out_v.at[:, pl.ds(_H, _NPW - _H)],
                             out_hbm.at[w, :, pl.ds(base + _H, _NPW - _H)], sem)
    cp_h1.wait()
    cp_h2.wait()


_sc_gather_built = None


def _sc_gather(sp, adjw):
    global _sc_gather_built
    if _sc_gather_built is None:
        mesh = plsc.VectorSubcoreMesh(core_axis_name="c", subcore_axis_name="s")
        cp = pltpu.CompilerParams()
        if "needs_layout_passes" in pltpu.CompilerParams.__dataclass_fields__:
            cp = dataclasses.replace(cp, needs_layout_passes=False)
        _sc_gather_built = functools.partial(
            pl.kernel,
            compiler_params=cp,
            out_type=jax.ShapeDtypeStruct((2, _K, _NSC), jnp.int32),
            mesh=mesh,
            scratch_types=[
                pltpu.VMEM((_NTC,), jnp.int32),
                pltpu.VMEM((_K, _NPW), jnp.int32),
                pltpu.VMEM((_K, _NPW), jnp.int32),
                pltpu.VMEM_SHARED((2, _NTC), jnp.int32),
                pltpu.SemaphoreType.DMA,
                pltpu.SemaphoreType.DMA,
            ],
        )(_sc_gather_body)
    return _sc_gather_built(sp, adjw)


def _mux_states(pm, L):
    D = [L[j] ^ lax.shift_right_logical(L[j], 1) for j in range(8)]
    outs = []
    for w in range(2):
        mk = [pm[w, k] for k in range(_K)]

        def res(lo, lev):
            if lev == 1:
                j, s5 = lo >> 5, lo & 31
                a = (L[j] << (31 - s5)) >> 31
                d = (D[j] << (31 - s5)) >> 31
                return a ^ (d & mk[0])
            a = res(lo, lev - 1)
            b = res(lo + (1 << (lev - 1)), lev - 1)
            return a ^ ((a ^ b) & mk[lev - 1])

        outs.append(res(0, 8))
    return jnp.stack(outs)


def _mux_body(planes_ref, lutp_ref, xp_ref, out_ref):
    L = [lutp_ref[j] for j in range(8)]
    pm = planes_ref[...].reshape(2, 8, 8, 128)
    out_ref[...] = _mux_states(pm, L).reshape(2, 1024)

    @pl.when(pl.program_id(0) == 0)
    def _():
        xq = jnp.concatenate(
            [xp_ref[...], jnp.zeros((2, 1024 - _BITS), jnp.int32)], axis=1)
        out_ref[...] = out_ref[...] ^ xq


def _mux_call(planes2, lutp3, xp_t):
    return pl.pallas_call(
        _mux_body,
        grid=(_NBLK,),
        in_specs=[
            pl.BlockSpec((2, 8, 1024), lambda i: (0, 0, i)),
            pl.BlockSpec((8, 8, 128), lambda i: (0, i, 0)),
            pl.BlockSpec((2, _BITS), lambda i: (0, 0)),
        ],
        out_specs=pl.BlockSpec((2, 1024), lambda i: (0, i)),
        out_shape=jax.ShapeDtypeStruct((2, _NTC), jnp.int32),
    )(planes2, lutp3, xp_t)


def _muxro_body(planes_ref, lutp_ref, wt_ref, b_ref, out_ref):
    L = [lutp_ref[j] for j in range(8)]
    pm = planes_ref[...].reshape(2, 8, 8, 128)
    sp = _mux_states(pm, L).reshape(2, 1024)

    sh = lax.broadcasted_iota(jnp.int32, (32, 1024), 0)
    f0 = (jnp.broadcast_to(sp[0:1], (32, 1024)) >> sh) & 1
    f1 = (jnp.broadcast_to(sp[1:2], (32, 1024)) >> sh) & 1
    feats = jnp.concatenate([f0, f1], axis=0).astype(jnp.float32)
    part = jnp.dot(feats, wt_ref[...], preferred_element_type=jnp.float32)

    @pl.when(pl.program_id(0) == 0)
    def _():
        out_ref[...] = jnp.broadcast_to(b_ref[...], (64, _OUTP))

    out_ref[...] += part

    @pl.when(pl.program_id(0) == _NBLK - 1)
    def _():
        out_ref[...] = jax.nn.sigmoid(out_ref[...])


def _muxro_call(planes2, lutp3, wt, bias):
    return pl.pallas_call(
        _muxro_body,
        grid=(_NBLK,),
        in_specs=[
            pl.BlockSpec((2, 8, 1024), lambda i: (0, 0, i)),
            pl.BlockSpec((8, 8, 128), lambda i: (0, i, 0)),
            pl.BlockSpec((1024, _OUTP), lambda i: (i, 0)),
            pl.BlockSpec((1, _OUTP), lambda i: (0, 0)),
        ],
        out_specs=pl.BlockSpec((64, _OUTP), lambda i: (0, 0)),
        out_shape=jax.ShapeDtypeStruct((64, _OUTP), jnp.float32),
    )(planes2, lutp3, wt, bias)


def kernel(x, adj_list, lut, init_state, W, b):
    lutT3 = jnp.pad(lut.T, ((0, 0), (0, _NTC - _N))).reshape(
        256, _NBLK * 8, 128)
    x3 = x.reshape(_B, _T * _BITS).reshape(_B, 8, 128)
    ini = jnp.pad(init_state, (0, _NTC - _N)).reshape(1, _NTC)
    adjw = jnp.pad(adj_list.T, ((0, 0), (0, _NSC - _N))).reshape(
        _K, 16, _NPW).transpose(1, 0, 2)
    wt = jnp.zeros((_NTC, _OUTP), jnp.float32).at[_BITS:_N, :_OUT].set(W.T)
    bias = jnp.zeros((1, _OUTP), jnp.float32).at[0, :_OUT].set(b)

    lutp3, xp, sp = _pack_call(lutT3, x3, ini)
    xp2 = xp.reshape(2, 1024).reshape(2, _T, _BITS)

    for t in range(_T):
        planes = _sc_gather(sp, adjw)
        if t < _T - 1:
            sp = _mux_call(planes, lutp3, xp2[:, t + 1, :])
        else:
            out = _muxro_call(planes, lutp3, wt, bias)

    return out[:, :_OUT]


# speedup vs baseline: 1.0098x; 1.0098x over previous
import dataclasses
import functools

import jax
import jax.numpy as jnp
from jax import lax
from jax.experimental import pallas as pl
from jax.experimental.pallas import tpu as pltpu
from jax.experimental.pallas import tpu_sc as plsc

_N = 10000
_K = 8
_BITS = 128
_B = 64
_T = 8
_OUT = 10

_NTC = 10240
_NBLK = _NTC // 1024
_NSC = 10240
_NW = 32
_NPW = _NSC // 16
_OUTP = 128


def _pack_body(lutT_ref, x_ref, ini_ref, lutp_ref, xp_ref, sp0_ref):
    rows = lutT_ref[...]
    outs = []
    for j in range(8):
        acc = rows[32 * j]
        for s in range(1, 32):
            acc = acc | (rows[32 * j + s] << s)
        outs.append(acc)
    lutp_ref[...] = jnp.stack(outs)

    neg = -ini_ref[...]
    val = jnp.broadcast_to(neg, (2, 1024))

    @pl.when(pl.program_id(0) == 0)
    def _():
        xr = x_ref[...]
        lo = xr[0]
        hi = xr[32]
        for bb in range(1, 32):
            lo = lo | (xr[bb] << bb)
            hi = hi | (xr[32 + bb] << bb)
        xp_ref[...] = jnp.stack([lo, hi])
        x0 = jnp.stack([lo[0], hi[0]])
        xq = jnp.concatenate([x0, jnp.zeros((2, 1024 - _BITS), jnp.int32)],
                             axis=1)
        sp0_ref[...] = val ^ xq

    @pl.when(pl.program_id(0) != 0)
    def _():
        sp0_ref[...] = val


def _pack_call(lutT3, x3, ini):
    return pl.pallas_call(
        _pack_body,
        grid=(_NBLK,),
        in_specs=[
            pl.BlockSpec((256, 8, 128), lambda i: (0, i, 0)),
            pl.BlockSpec((64, 8, 128), lambda i: (0, 0, 0)),
            pl.BlockSpec((1, 1024), lambda i: (0, i)),
        ],
        out_specs=[
            pl.BlockSpec((8, 8, 128), lambda i: (0, i, 0)),
            pl.BlockSpec((2, 8, 128), lambda i: (0, 0, 0)),
            pl.BlockSpec((2, 1024), lambda i: (0, i)),
        ],
        out_shape=[
            jax.ShapeDtypeStruct((8, _NBLK * 8, 128), jnp.int32),
            jax.ShapeDtypeStruct((2, 8, 128), jnp.int32),
            jax.ShapeDtypeStruct((2, _NTC), jnp.int32),
        ],
    )(lutT3, x3, ini)


def _sc_gather_body(sp_hbm, adj_hbm, out_hbm, sp_v, adj_v, out_v, sp_sh, sem,
                    sem_a):
    c = lax.axis_index("c")
    s = lax.axis_index("s")
    wid = s * 2 + c
    w = wid & 1
    nslice = wid >> 1
    base = nslice * _NPW
    cp_adj = pltpu.async_copy(adj_hbm.at[nslice], adj_v, sem_a)

    @pl.when(s == 0)
    def _():
        pltpu.async_copy(sp_hbm, sp_sh, sem).wait()

    plsc.subcore_barrier()
    pltpu.async_copy(sp_sh.at[w], sp_v, sem).wait()
    cp_adj.wait()

    @plsc.parallel_loop(0, _NPW, step=16, unroll=4)
    def _(g):
        for k in range(_K):
            idx = adj_v[k, pl.ds(g, 16)]
            out_v[k, pl.ds(g, 16)] = plsc.load_gather(sp_v, [idx])

    pltpu.async_copy(out_v, out_hbm.at[w, :, pl.ds(base, _NPW)], sem).wait()


_sc_gather_built = None


def _sc_gather(sp, adjw):
    global _sc_gather_built
    if _sc_gather_built is None:
        mesh = plsc.VectorSubcoreMesh(core_axis_name="c", subcore_axis_name="s")
        cp = pltpu.CompilerParams()
        if "needs_layout_passes" in pltpu.CompilerParams.__dataclass_fields__:
            cp = dataclasses.replace(cp, needs_layout_passes=False)
        _sc_gather_built = functools.partial(
            pl.kernel,
            compiler_params=cp,
            out_type=jax.ShapeDtypeStruct((2, _K, _NSC), jnp.int32),
            mesh=mesh,
            scratch_types=[
                pltpu.VMEM((_NTC,), jnp.int32),
                pltpu.VMEM((_K, _NPW), jnp.int32),
                pltpu.VMEM((_K, _NPW), jnp.int32),
                pltpu.VMEM_SHARED((2, _NTC), jnp.int32),
                pltpu.SemaphoreType.DMA,
                pltpu.SemaphoreType.DMA,
            ],
        )(_sc_gather_body)
    return _sc_gather_built(sp, adjw)


def _mux_states(pm, L):
    D = [L[j] ^ lax.shift_right_logical(L[j], 1) for j in range(8)]
    outs = []
    for w in range(2):
        mk = [pm[w, k] for k in range(_K)]

        def res(lo, lev):
            if lev == 1:
                j, s5 = lo >> 5, lo & 31
                a = (L[j] << (31 - s5)) >> 31
                d = (D[j] << (31 - s5)) >> 31
                return a ^ (d & mk[0])
            a = res(lo, lev - 1)
            b = res(lo + (1 << (lev - 1)), lev - 1)
            return a ^ ((a ^ b) & mk[lev - 1])

        outs.append(res(0, 8))
    return jnp.stack(outs)


def _mux_body(planes_ref, lutp_ref, xp_ref, out_ref):
    L = [lutp_ref[j] for j in range(8)]
    pm = planes_ref[...].reshape(2, 8, 8, 128)
    out_ref[...] = _mux_states(pm, L).reshape(2, 1024)

    @pl.when(pl.program_id(0) == 0)
    def _():
        xq = jnp.concatenate(
            [xp_ref[...], jnp.zeros((2, 1024 - _BITS), jnp.int32)], axis=1)
        out_ref[...] = out_ref[...] ^ xq


def _mux_call(planes2, lutp3, xp_t):
    return pl.pallas_call(
        _mux_body,
        grid=(_NBLK,),
        in_specs=[
            pl.BlockSpec((2, 8, 1024), lambda i: (0, 0, i)),
            pl.BlockSpec((8, 8, 128), lambda i: (0, i, 0)),
            pl.BlockSpec((2, _BITS), lambda i: (0, 0)),
        ],
        out_specs=pl.BlockSpec((2, 1024), lambda i: (0, i)),
        out_shape=jax.ShapeDtypeStruct((2, _NTC), jnp.int32),
    )(planes2, lutp3, xp_t)


def _muxro_body(planes_ref, lutp_ref, wt_ref, b_ref, out_ref):
    L = [lutp_ref[j] for j in range(8)]
    pm = planes_ref[...].reshape(2, 8, 8, 128)
    sp = _mux_states(pm, L).reshape(2, 1024)

    sh = lax.broadcasted_iota(jnp.int32, (32, 1024), 0)
    f0 = (jnp.broadcast_to(sp[0:1], (32, 1024)) >> sh) & 1
    f1 = (jnp.broadcast_to(sp[1:2], (32, 1024)) >> sh) & 1
    feats = jnp.concatenate([f0, f1], axis=0).astype(jnp.float32)
    part = jnp.dot(feats, wt_ref[...], preferred_element_type=jnp.float32)

    @pl.when(pl.program_id(0) == 0)
    def _():
        out_ref[...] = jnp.broadcast_to(b_ref[...], (64, _OUTP))

    out_ref[...] += part

    @pl.when(pl.program_id(0) == _NBLK - 1)
    def _():
        out_ref[...] = jax.nn.sigmoid(out_ref[...])


def _muxro_call(planes2, lutp3, wt, bias):
    return pl.pallas_call(
        _muxro_body,
        grid=(_NBLK,),
        in_specs=[
            pl.BlockSpec((2, 8, 1024), lambda i: (0, 0, i)),
            pl.BlockSpec((8, 8, 128), lambda i: (0, i, 0)),
            pl.BlockSpec((1024, _OUTP), lambda i: (i, 0)),
            pl.BlockSpec((1, _OUTP), lambda i: (0, 0)),
        ],
        out_specs=pl.BlockSpec((64, _OUTP), lambda i: (0, 0)),
        out_shape=jax.ShapeDtypeStruct((64, _OUTP), jnp.float32),
    )(planes2, lutp3, wt, bias)


def kernel(x, adj_list, lut, init_state, W, b):
    lutT3 = jnp.pad(lut.T, ((0, 0), (0, _NTC - _N))).reshape(
        256, _NBLK * 8, 128)
    x3 = x.reshape(_B, _T * _BITS).reshape(_B, 8, 128)
    ini = jnp.pad(init_state, (0, _NTC - _N)).reshape(1, _NTC)
    adjw = jnp.pad(adj_list.T, ((0, 0), (0, _NSC - _N))).reshape(
        _K, 16, _NPW).transpose(1, 0, 2)
    wt = jnp.zeros((_NTC, _OUTP), jnp.float32).at[_BITS:_N, :_OUT].set(W.T)
    bias = jnp.zeros((1, _OUTP), jnp.float32).at[0, :_OUT].set(b)

    lutp3, xp, sp = _pack_call(lutT3, x3, ini)
    xp2 = xp.reshape(2, 1024).reshape(2, _T, _BITS)

    for t in range(_T):
        planes = _sc_gather(sp, adjw)
        if t < _T - 1:
            sp = _mux_call(planes, lutp3, xp2[:, t + 1, :])
        else:
            out = _muxro_call(planes, lutp3, wt, bias)

    return out[:, :_OUT]
